# baseline (speedup 1.0000x reference)
import functools

import jax
import jax.numpy as jnp
from jax import lax
from jax.experimental import pallas as pl
from jax.experimental.pallas import tpu as pltpu
from jax.experimental.pallas import tpu_sc as plsc

N = 10000
E = 320000
F = 128

NC = 2
NS = 16
NW = NC * NS
K = 128
NCH = 80
CPW = 85
SLABS = CPW + 2
EPWP = NCH * K
NPAD = 10240
RPS = NPAD // NS

_mesh = plsc.VectorSubcoreMesh(core_axis_name="c", subcore_axis_name="s")


@functools.partial(
    pl.kernel,
    out_type=jax.ShapeDtypeStruct((NC, NPAD, F), jnp.float32),
    mesh=_mesh,
    scratch_types=[
        pltpu.VMEM((3, 2, K), jnp.int32),
        pltpu.VMEM((3 * K,), jnp.float32),
        pltpu.VMEM((2, K, F), jnp.float32),
        pltpu.VMEM_SHARED((NPAD, F), jnp.float32),
        pltpu.SemaphoreType.DMA,
        pltpu.SemaphoreType.DMA,
        pltpu.SemaphoreType.DMA,
        pltpu.SemaphoreType.DMA,
        pltpu.SemaphoreType.DMA,
        pltpu.SemaphoreType.DMA,
        pltpu.SemaphoreType.DMA,
    ],
    compiler_params=pltpu.CompilerParams(needs_layout_passes=False),
)
def _spmm_sc(sup_hbm, pk_hbm, w_hbm, zer_hbm, out_hbm, pk_v, w_v, rows_v, acc,
             pk_sem0, pk_sem1, pk_sem2, g_sem0, g_sem1, s_sem0, s_sem1):
    pk_sem = (pk_sem0, pk_sem1, pk_sem2)
    g_sem = (g_sem0, g_sem1)
    s_sem = (s_sem0, s_sem1)
    c = lax.axis_index("c")
    s = lax.axis_index("s")
    wid = c * NS + s
    sbase = wid * SLABS


    def start_pk(j, p):
        pltpu.async_copy(pk_hbm.at[sbase + j], pk_v.at[p], pk_sem[p])
        pltpu.async_copy(w_hbm.at[sbase + j], w_v.at[pl.ds(p * K, K)],
                         pk_sem[p])

    def wait_pk(p):
        pltpu.make_async_copy(pk_hbm.at[0], pk_v.at[p], pk_sem[p]).wait()
        pltpu.make_async_copy(w_hbm.at[0], w_v.at[pl.ds(p * K, K)],
                              pk_sem[p]).wait()

    def start_gather(b, p):
        pltpu.async_copy(sup_hbm.at[pk_v.at[p, 0]], rows_v.at[b], g_sem[b])

    def wait_gather(b, p):
        pltpu.make_async_copy(sup_hbm.at[pk_v.at[p, 0]], rows_v.at[b],
                              g_sem[b]).wait()

    def start_scatter(b, p):
        pltpu.async_copy(rows_v.at[b], acc.at[pk_v.at[p, 1]], s_sem[b],
                         add=True)

    def wait_scatter(b, p):
        pltpu.make_async_copy(rows_v.at[b], acc.at[pk_v.at[p, 1]],
                              s_sem[b]).wait()

    def scale_rows(b, p):
        def body(i, carry):
            w = plsc.load_gather(w_v, [jnp.full((16,), p * K, jnp.int32) + i])
            for f in range(F // 16):
                rows_v[b, i, pl.ds(f * 16, 16)] = (
                    rows_v[b, i, pl.ds(f * 16, 16)] * w)
            return carry

        lax.fori_loop(0, K, body, 0, unroll=4)

    def process(j, b, p, first=False):
        nb, np_, rp = 1 - b, (p + 1) % 3, (p + 2) % 3
        wait_gather(b, p)
        if not first:
            wait_scatter(nb, rp)
        start_pk(j + 2, rp)
        wait_pk(np_)
        start_gather(nb, np_)
        scale_rows(b, p)
        start_scatter(b, p)

    pltpu.sync_copy(zer_hbm, acc.at[pl.ds(s * RPS, RPS)])
    plsc.subcore_barrier()

    start_pk(0, 0)
    start_pk(1, 1)
    wait_pk(0)
    start_gather(0, 0)

    process(0, 0, 0, first=True)

    def outer(g, carry):
        for t in range(6):
            process(1 + 6 * g + t, (1 + t) % 2, (1 + t) % 3)
        return carry

    lax.fori_loop(0, (CPW - 1) // 6, outer, 0)

    wait_gather(CPW % 2, CPW % 3)
    wait_pk((CPW + 1) % 3)
    wait_scatter((CPW - 1) % 2, (CPW - 1) % 3)
    plsc.subcore_barrier()
    pltpu.sync_copy(acc.at[pl.ds(s * RPS, RPS)],
                    out_hbm.at[c].at[pl.ds(s * RPS, RPS)])


def _pack_edges(src, dst, w):
    pad = NW * EPWP - E
    srcp = jnp.pad(src, (0, pad)).reshape(NW, NCH, K)
    dstp = jnp.pad(dst, (0, pad)).reshape(NW, NCH, K)
    pk = jnp.stack([srcp, dstp], axis=2)
    pk = jnp.pad(pk, ((0, 0), (0, SLABS - NCH), (0, 0), (0, 0)))
    wp = jnp.pad(w, (0, pad)).reshape(NW, NCH, K)
    wp = jnp.pad(wp, ((0, 0), (0, SLABS - NCH), (0, 0)))
    return pk.reshape(NW * SLABS, 2, K), wp.reshape(NW * SLABS, K)


def _mm_body(x_ref, w_ref, o_ref):
    o_ref[...] = jnp.dot(x_ref[...], w_ref[...],
                         preferred_element_type=jnp.float32)


def _mm(x, W, bm=1000):
    m = x.shape[0]
    return pl.pallas_call(
        _mm_body,
        grid=(m // bm,),
        in_specs=[pl.BlockSpec((bm, F), lambda i: (i, 0)),
                  pl.BlockSpec((F, F), lambda i: (0, 0))],
        out_specs=pl.BlockSpec((bm, F), lambda i: (i, 0)),
        out_shape=jax.ShapeDtypeStruct((m, F), jnp.float32),
    )(x, W)


def _mid_body(p_ref, b_ref, w_ref, o_ref):
    h = jnp.maximum(p_ref[0] + p_ref[1] + b_ref[...], 0.0)
    o_ref[...] = jnp.dot(h, w_ref[...], preferred_element_type=jnp.float32)


def _mid(p, b1, W2, bm=1000):
    return pl.pallas_call(
        _mid_body,
        grid=(N // bm,),
        in_specs=[pl.BlockSpec((NC, bm, F), lambda i: (0, i, 0)),
                  pl.BlockSpec((1, F), lambda i: (0, 0)),
                  pl.BlockSpec((F, F), lambda i: (0, 0))],
        out_specs=pl.BlockSpec((bm, F), lambda i: (i, 0)),
        out_shape=jax.ShapeDtypeStruct((N, F), jnp.float32),
    )(p, b1.reshape(1, F), W2)


def _fin_body(p_ref, b_ref, o_ref):
    o_ref[...] = p_ref[0] + p_ref[1] + b_ref[...]


def _fin(p, b2, bm=1000):
    return pl.pallas_call(
        _fin_body,
        grid=(N // bm,),
        in_specs=[pl.BlockSpec((NC, bm, F), lambda i: (0, i, 0)),
                  pl.BlockSpec((1, F), lambda i: (0, 0))],
        out_specs=pl.BlockSpec((bm, F), lambda i: (i, 0)),
        out_shape=jax.ShapeDtypeStruct((N, F), jnp.float32),
    )(p, b2.reshape(1, F))


def kernel(x, edge_index, edge_weight, W1, b1, W2, b2):
    pk, pw = _pack_edges(edge_index[0], edge_index[1], edge_weight)
    zer = jnp.zeros((RPS, F), dtype=jnp.float32)

    support1 = _mm(x, W1)
    p1 = _spmm_sc(support1, pk, pw, zer)
    support2 = _mid(p1, b1, W2)
    p2 = _spmm_sc(support2, pk, pw, zer)
    return _fin(p2, b2)


# speedup vs baseline: 1.4417x; 1.4417x over previous
import functools

import jax
import jax.numpy as jnp
from jax import lax
from jax.experimental import pallas as pl
from jax.experimental.pallas import tpu as pltpu
from jax.experimental.pallas import tpu_sc as plsc

N = 10000
E = 320000
F = 128

NC = 2
NS = 16
NW = NC * NS
K = 128
NCH = 80
CPW = 82
SLABS = CPW + 2
EPWP = NCH * K
NPAD = 10240
RPS = NPAD // NS

_mesh = plsc.VectorSubcoreMesh(core_axis_name="c", subcore_axis_name="s")


@functools.partial(
    pl.kernel,
    out_type=jax.ShapeDtypeStruct((NC, NPAD, F), jnp.float32),
    mesh=_mesh,
    scratch_types=[
        pltpu.VMEM((2, 2, K), jnp.int32),
        pltpu.VMEM((2 * K,), jnp.float32),
        pltpu.VMEM((2, K), jnp.int32),
        pltpu.VMEM((2, K, F), jnp.float32),
        pltpu.VMEM_SHARED((NPAD, F), jnp.float32),
        pltpu.SemaphoreType.DMA,
        pltpu.SemaphoreType.DMA,
        pltpu.SemaphoreType.DMA,
        pltpu.SemaphoreType.DMA,
        pltpu.SemaphoreType.DMA,
        pltpu.SemaphoreType.DMA,
    ],
    compiler_params=pltpu.CompilerParams(needs_layout_passes=False),
)
def _spmm_sc(sup_hbm, pk_hbm, w_hbm, zer_hbm, out_hbm, pk_v, w_v, dsti_v,
             rows_v, acc, pk_sem0, pk_sem1, g_sem0, g_sem1, s_sem0, s_sem1):
    pk_sem = (pk_sem0, pk_sem1)
    g_sem = (g_sem0, g_sem1)
    s_sem = (s_sem0, s_sem1)
    c = lax.axis_index("c")
    s = lax.axis_index("s")
    wid = c * NS + s
    sbase = wid * SLABS


    def start_pk(j, b):
        pltpu.async_copy(pk_hbm.at[sbase + j], pk_v.at[b], pk_sem[b])
        pltpu.async_copy(w_hbm.at[sbase + j], w_v.at[pl.ds(b * K, K)],
                         pk_sem[b])

    def wait_pk(b):
        pltpu.make_async_copy(pk_hbm.at[0], pk_v.at[b], pk_sem[b]).wait()
        pltpu.make_async_copy(w_hbm.at[0], w_v.at[pl.ds(b * K, K)],
                              pk_sem[b]).wait()

    def start_gather(b):
        pltpu.async_copy(sup_hbm.at[pk_v.at[b, 0]], rows_v.at[b], g_sem[b])

    def wait_gather(b):
        pltpu.make_async_copy(sup_hbm.at[pk_v.at[b, 0]], rows_v.at[b],
                              g_sem[b]).wait()

    def start_scatter(b):
        pltpu.async_copy(rows_v.at[b], acc.at[dsti_v.at[b]], s_sem[b],
                         add=True)

    def wait_scatter(b):
        pltpu.make_async_copy(rows_v.at[b], acc.at[dsti_v.at[b]],
                              s_sem[b]).wait()

    def copy_dst(b):
        for i in range(K // 16):
            dsti_v[b, pl.ds(i * 16, 16)] = pk_v[b, 1, pl.ds(i * 16, 16)]

    def scale_rows(b):
        def body(i, carry):
            w = plsc.load_gather(w_v, [jnp.full((16,), b * K, jnp.int32) + i])
            for f in range(F // 16):
                rows_v[b, i, pl.ds(f * 16, 16)] = (
                    rows_v[b, i, pl.ds(f * 16, 16)] * w)
            return carry

        lax.fori_loop(0, K, body, 0, unroll=2)

    def process(j, b, first=False):
        nb = 1 - b
        wait_gather(b)
        if not first:
            wait_scatter(nb)
        wait_pk(nb)
        start_gather(nb)
        copy_dst(b)
        scale_rows(b)
        start_pk(j + 2, b)
        start_scatter(b)

    pltpu.sync_copy(zer_hbm, acc.at[pl.ds(s * RPS, RPS)])
    plsc.subcore_barrier()

    start_pk(0, 0)
    start_pk(1, 1)
    wait_pk(0)
    start_gather(0)

    process(0, 0, first=True)
    process(1, 1)

    def outer(g, carry):
        process(2 * g, 0)
        process(2 * g + 1, 1)
        return carry

    lax.fori_loop(1, CPW // 2, outer, 0)

    wait_gather(CPW % 2)
    wait_pk((CPW + 1) % 2)
    wait_scatter((CPW - 1) % 2)
    plsc.subcore_barrier()
    pltpu.sync_copy(acc.at[pl.ds(s * RPS, RPS)],
                    out_hbm.at[c].at[pl.ds(s * RPS, RPS)])


def _pack_edges(src, dst, w):
    pad = NW * EPWP - E
    srcp = jnp.pad(src, (0, pad)).reshape(NW, NCH, K)
    dstp = jnp.pad(dst, (0, pad)).reshape(NW, NCH, K)
    pk = jnp.stack([srcp, dstp], axis=2)
    pk = jnp.pad(pk, ((0, 0), (0, SLABS - NCH), (0, 0), (0, 0)))
    wp = jnp.pad(w, (0, pad)).reshape(NW, NCH, K)
    wp = jnp.pad(wp, ((0, 0), (0, SLABS - NCH), (0, 0)))
    return pk.reshape(NW * SLABS, 2, K), wp.reshape(NW * SLABS, K)


def _mm_body(x_ref, w_ref, o_ref):
    o_ref[...] = jnp.dot(x_ref[...], w_ref[...],
                         preferred_element_type=jnp.float32)


def _mm(x, W, bm=1000):
    m = x.shape[0]
    return pl.pallas_call(
        _mm_body,
        grid=(m // bm,),
        in_specs=[pl.BlockSpec((bm, F), lambda i: (i, 0)),
                  pl.BlockSpec((F, F), lambda i: (0, 0))],
        out_specs=pl.BlockSpec((bm, F), lambda i: (i, 0)),
        out_shape=jax.ShapeDtypeStruct((m, F), jnp.float32),
    )(x, W)


def _mid_body(p_ref, b_ref, w_ref, o_ref):
    h = jnp.maximum(p_ref[0] + p_ref[1] + b_ref[...], 0.0)
    o_ref[...] = jnp.dot(h, w_ref[...], preferred_element_type=jnp.float32)


def _mid(p, b1, W2, bm=1000):
    return pl.pallas_call(
        _mid_body,
        grid=(N // bm,),
        in_specs=[pl.BlockSpec((NC, bm, F), lambda i: (0, i, 0)),
                  pl.BlockSpec((1, F), lambda i: (0, 0)),
                  pl.BlockSpec((F, F), lambda i: (0, 0))],
        out_specs=pl.BlockSpec((bm, F), lambda i: (i, 0)),
        out_shape=jax.ShapeDtypeStruct((N, F), jnp.float32),
    )(p, b1.reshape(1, F), W2)


def _fin_body(p_ref, b_ref, o_ref):
    o_ref[...] = p_ref[0] + p_ref[1] + b_ref[...]


def _fin(p, b2, bm=1000):
    return pl.pallas_call(
        _fin_body,
        grid=(N // bm,),
        in_specs=[pl.BlockSpec((NC, bm, F), lambda i: (0, i, 0)),
                  pl.BlockSpec((1, F), lambda i: (0, 0))],
        out_specs=pl.BlockSpec((bm, F), lambda i: (i, 0)),
        out_shape=jax.ShapeDtypeStruct((N, F), jnp.float32),
    )(p, b2.reshape(1, F))


def kernel(x, edge_index, edge_weight, W1, b1, W2, b2):
    pk, pw = _pack_edges(edge_index[0], edge_index[1], edge_weight)
    zer = jnp.zeros((RPS, F), dtype=jnp.float32)

    support1 = _mm(x, W1)
    p1 = _spmm_sc(support1, pk, pw, zer)
    support2 = _mid(p1, b1, W2)
    p2 = _spmm_sc(support2, pk, pw, zer)
    return _fin(p2, b2)


# speedup vs baseline: 2.1564x; 1.4957x over previous
import functools

import jax
import jax.numpy as jnp
from jax import lax
from jax.experimental import pallas as pl
from jax.experimental.pallas import tpu as pltpu
from jax.experimental.pallas import tpu_sc as plsc

N = 10000
E = 320000
F = 128

NC = 2
NS = 16
NW = NC * NS
K = 128
NCH = 80
CPW = 82
SLABS = CPW + 2
EPWP = NCH * K
NPAD = 10240
RPS = NPAD // NS

_mesh = plsc.VectorSubcoreMesh(core_axis_name="c", subcore_axis_name="s")


@functools.partial(
    pl.kernel,
    out_type=jax.ShapeDtypeStruct((NC, NPAD, F), jnp.float32),
    mesh=_mesh,
    scratch_types=[
        pltpu.VMEM((2, 2, K), jnp.int32),
        pltpu.VMEM((2 * K,), jnp.float32),
        pltpu.VMEM((2, K), jnp.int32),
        pltpu.VMEM((2, K, F), jnp.float32),
        pltpu.VMEM_SHARED((NPAD, F), jnp.float32),
        pltpu.SemaphoreType.DMA,
        pltpu.SemaphoreType.DMA,
        pltpu.SemaphoreType.DMA,
        pltpu.SemaphoreType.DMA,
        pltpu.SemaphoreType.DMA,
        pltpu.SemaphoreType.DMA,
    ],
    compiler_params=pltpu.CompilerParams(needs_layout_passes=False),
)
def _spmm_sc(sup_hbm, pk_hbm, w_hbm, zer_hbm, out_hbm, pk_v, w_v, dsti_v,
             rows_v, acc, pk_sem0, pk_sem1, g_sem0, g_sem1, s_sem0, s_sem1):
    pk_sem = (pk_sem0, pk_sem1)
    g_sem = (g_sem0, g_sem1)
    s_sem = (s_sem0, s_sem1)
    c = lax.axis_index("c")
    s = lax.axis_index("s")
    wid = c * NS + s
    sbase = wid * SLABS


    def start_pk(j, b):
        pltpu.async_copy(pk_hbm.at[sbase + j], pk_v.at[b], pk_sem[b])
        pltpu.async_copy(w_hbm.at[sbase + j], w_v.at[pl.ds(b * K, K)],
                         pk_sem[b])

    def wait_pk(b):
        pltpu.make_async_copy(pk_hbm.at[0], pk_v.at[b], pk_sem[b]).wait()
        pltpu.make_async_copy(w_hbm.at[0], w_v.at[pl.ds(b * K, K)],
                              pk_sem[b]).wait()

    def start_gather(b):
        pltpu.async_copy(sup_hbm.at[pk_v.at[b, 0]], rows_v.at[b], g_sem[b])

    def wait_gather(b):
        pltpu.make_async_copy(sup_hbm.at[pk_v.at[b, 0]], rows_v.at[b],
                              g_sem[b]).wait()

    def start_scatter(b):
        pltpu.async_copy(rows_v.at[b], acc.at[dsti_v.at[b]], s_sem[b],
                         add=True)

    def wait_scatter(b):
        pltpu.make_async_copy(rows_v.at[b], acc.at[dsti_v.at[b]],
                              s_sem[b]).wait()

    def copy_dst(b):
        for i in range(K // 16):
            dsti_v[b, pl.ds(i * 16, 16)] = pk_v[b, 1, pl.ds(i * 16, 16)]

    def scale_rows(b):
        def body(i, carry):
            w = plsc.load_gather(w_v, [jnp.full((16,), b * K, jnp.int32) + i])
            for f in range(F // 16):
                rows_v[b, i, pl.ds(f * 16, 16)] = (
                    rows_v[b, i, pl.ds(f * 16, 16)] * w)
            return carry

        lax.fori_loop(0, K, body, 0, unroll=2)

    pltpu.sync_copy(zer_hbm, acc.at[pl.ds(s * RPS, RPS)])
    plsc.subcore_barrier()

    def chunk(j, carry):
        d1 = pltpu.async_copy(pk_hbm.at[sbase + j], pk_v.at[0], pk_sem[0])
        d2 = pltpu.async_copy(w_hbm.at[sbase + j], w_v.at[pl.ds(0, K)],
                              pk_sem[0])
        d1.wait()
        d2.wait()
        dg = pltpu.async_copy(sup_hbm.at[pk_v.at[0, 0]], rows_v.at[0],
                              g_sem[0])
        dg.wait()
        scale_rows(0)
        ds = pltpu.async_copy(rows_v.at[0], acc.at[pk_v.at[0, 1]], s_sem[0],
                              add=True)
        ds.wait()
        return carry

    lax.fori_loop(0, NCH, chunk, 0)
    plsc.subcore_barrier()
    pltpu.sync_copy(acc.at[pl.ds(s * RPS, RPS)],
                    out_hbm.at[c].at[pl.ds(s * RPS, RPS)])


def _pack_edges(src, dst, w):
    pad = NW * EPWP - E
    srcp = jnp.pad(src, (0, pad)).reshape(NW, NCH, K)
    dstp = jnp.pad(dst, (0, pad)).reshape(NW, NCH, K)
    pk = jnp.stack([srcp, dstp], axis=2)
    pk = jnp.pad(pk, ((0, 0), (0, SLABS - NCH), (0, 0), (0, 0)))
    wp = jnp.pad(w, (0, pad)).reshape(NW, NCH, K)
    wp = jnp.pad(wp, ((0, 0), (0, SLABS - NCH), (0, 0)))
    return pk.reshape(NW * SLABS, 2, K), wp.reshape(NW * SLABS, K)


def _mm_body(x_ref, w_ref, o_ref):
    o_ref[...] = jnp.dot(x_ref[...], w_ref[...],
                         preferred_element_type=jnp.float32)


def _mm(x, W, bm=1000):
    m = x.shape[0]
    return pl.pallas_call(
        _mm_body,
        grid=(m // bm,),
        in_specs=[pl.BlockSpec((bm, F), lambda i: (i, 0)),
                  pl.BlockSpec((F, F), lambda i: (0, 0))],
        out_specs=pl.BlockSpec((bm, F), lambda i: (i, 0)),
        out_shape=jax.ShapeDtypeStruct((m, F), jnp.float32),
    )(x, W)


def _mid_body(p_ref, b_ref, w_ref, o_ref):
    h = jnp.maximum(p_ref[0] + p_ref[1] + b_ref[...], 0.0)
    o_ref[...] = jnp.dot(h, w_ref[...], preferred_element_type=jnp.float32)


def _mid(p, b1, W2, bm=1000):
    return pl.pallas_call(
        _mid_body,
        grid=(N // bm,),
        in_specs=[pl.BlockSpec((NC, bm, F), lambda i: (0, i, 0)),
                  pl.BlockSpec((1, F), lambda i: (0, 0)),
                  pl.BlockSpec((F, F), lambda i: (0, 0))],
        out_specs=pl.BlockSpec((bm, F), lambda i: (i, 0)),
        out_shape=jax.ShapeDtypeStruct((N, F), jnp.float32),
    )(p, b1.reshape(1, F), W2)


def _fin_body(p_ref, b_ref, o_ref):
    o_ref[...] = p_ref[0] + p_ref[1] + b_ref[...]


def _fin(p, b2, bm=1000):
    return pl.pallas_call(
        _fin_body,
        grid=(N // bm,),
        in_specs=[pl.BlockSpec((NC, bm, F), lambda i: (0, i, 0)),
                  pl.BlockSpec((1, F), lambda i: (0, 0))],
        out_specs=pl.BlockSpec((bm, F), lambda i: (i, 0)),
        out_shape=jax.ShapeDtypeStruct((N, F), jnp.float32),
    )(p, b2.reshape(1, F))


def kernel(x, edge_index, edge_weight, W1, b1, W2, b2):
    pk, pw = _pack_edges(edge_index[0], edge_index[1], edge_weight)
    zer = jnp.zeros((RPS, F), dtype=jnp.float32)

    support1 = _mm(x, W1)
    p1 = _spmm_sc(support1, pk, pw, zer)
    support2 = _mid(p1, b1, W2)
    p2 = _spmm_sc(support2, pk, pw, zer)
    return _fin(p2, b2)
